# baseline (speedup 1.0000x reference)
import functools
import jax
import jax.numpy as jnp
from jax import lax
from jax.experimental import pallas as pl
from jax.experimental.pallas import tpu as pltpu
from jax.experimental.pallas import tpu_sc as plsc

_N = 10000
_NP = 10240
_D = 512
_CW = 128
_NBLK = _D // _CW
_NC = 2
_NS = 16
_SLICE = _NP // _NS
_K = 80

_BM = 1000


def _first_body(x_ref, w_ref, dn_ref, o_ref):
    xw = jnp.dot(x_ref[...], w_ref[...], preferred_element_type=jnp.float32)
    dn = dn_ref[...]
    for b in range(_NBLK):
        o_ref[b] = dn * xw[:, b * _CW:(b + 1) * _CW]


def _first_matmul(x, W, dn):
    M, K = x.shape
    return pl.pallas_call(
        _first_body,
        grid=(M // _BM,),
        in_specs=[
            pl.BlockSpec((_BM, K), lambda i: (i, 0)),
            pl.BlockSpec((K, _D), lambda i: (0, 0)),
            pl.BlockSpec((_BM, 1), lambda i: (i, 0)),
        ],
        out_specs=pl.BlockSpec((_NBLK, _BM, _CW), lambda i: (0, i, 0)),
        out_shape=jax.ShapeDtypeStruct((_NBLK, _NP, _CW), jnp.float32),
    )(x, W, dn)


def _mid_body(agg_ref, u_ref, dn_ref, b_ref, w_ref, o_ref):
    acc = jnp.zeros((_BM, _D), dtype=jnp.float32)
    dn = dn_ref[...]
    for b in range(_NBLK):
        sl = slice(b * _CW, (b + 1) * _CW)
        h = jnp.maximum(dn * (agg_ref[b] + u_ref[b]) + b_ref[:, sl], 0.0)
        acc = acc + jnp.dot(h, w_ref[sl, :], preferred_element_type=jnp.float32)
    for b in range(_NBLK):
        o_ref[b] = dn * acc[:, b * _CW:(b + 1) * _CW]


def _mid_matmul(agg, u, dn, bias, W):
    M = dn.shape[0]
    return pl.pallas_call(
        _mid_body,
        grid=(M // _BM,),
        in_specs=[
            pl.BlockSpec((_NBLK, _BM, _CW), lambda i: (0, i, 0)),
            pl.BlockSpec((_NBLK, _BM, _CW), lambda i: (0, i, 0)),
            pl.BlockSpec((_BM, 1), lambda i: (i, 0)),
            pl.BlockSpec((1, _D), lambda i: (0, 0)),
            pl.BlockSpec((_D, _D), lambda i: (0, 0)),
        ],
        out_specs=pl.BlockSpec((_NBLK, _BM, _CW), lambda i: (0, i, 0)),
        out_shape=jax.ShapeDtypeStruct((_NBLK, _NP, _CW), jnp.float32),
    )(agg, u, dn, bias.reshape(1, _D), W)


def _fin_body(agg_ref, u_ref, dn_ref, b3_ref, wp1_ref, bp1_ref, wp2_ref,
              bp2_ref, emb_ref, z_ref):
    dn = dn_ref[...]
    cols = []
    for b in range(_NBLK):
        sl = slice(b * _CW, (b + 1) * _CW)
        cols.append(dn * (agg_ref[b] + u_ref[b]) + b3_ref[:, sl])
    emb = jnp.concatenate(cols, axis=1)
    emb_ref[...] = emb
    t = jnp.maximum(
        jnp.dot(emb, wp1_ref[...], preferred_element_type=jnp.float32)
        + bp1_ref[...], 0.0)
    z_ref[...] = jnp.dot(t, wp2_ref[...],
                         preferred_element_type=jnp.float32) + bp2_ref[...]


def _final_stage(agg, u, dn, b3, Wp1, bp1, Wp2, bp2):
    M = dn.shape[0]
    P = Wp1.shape[1]
    return pl.pallas_call(
        _fin_body,
        grid=(M // _BM,),
        in_specs=[
            pl.BlockSpec((_NBLK, _BM, _CW), lambda i: (0, i, 0)),
            pl.BlockSpec((_NBLK, _BM, _CW), lambda i: (0, i, 0)),
            pl.BlockSpec((_BM, 1), lambda i: (i, 0)),
            pl.BlockSpec((1, _D), lambda i: (0, 0)),
            pl.BlockSpec((_D, P), lambda i: (0, 0)),
            pl.BlockSpec((1, P), lambda i: (0, 0)),
            pl.BlockSpec((P, P), lambda i: (0, 0)),
            pl.BlockSpec((1, P), lambda i: (0, 0)),
        ],
        out_specs=[
            pl.BlockSpec((_BM, _D), lambda i: (i, 0)),
            pl.BlockSpec((_BM, P), lambda i: (i, 0)),
        ],
        out_shape=[
            jax.ShapeDtypeStruct((M, _D), jnp.float32),
            jax.ShapeDtypeStruct((M, P), jnp.float32),
        ],
    )(agg, u, dn, b3.reshape(1, _D), Wp1, bp1.reshape(1, P), Wp2,
      bp2.reshape(1, P))


def _spmm_sc(xwb, rr, cc, vn, zrows):
    E = rr.shape[0]
    epw = E // _NS
    chunks = epw // _K
    assert chunks * _K == epw
    NB = 4
    assert chunks % NB == 0
    M = chunks // NB
    mesh = plsc.VectorSubcoreMesh(core_axis_name="c", subcore_axis_name="s",
                                  num_cores=_NC, num_subcores=_NS)
    bpc = _NBLK // _NC

    @functools.partial(
        pl.kernel,
        out_type=jax.ShapeDtypeStruct((_NBLK * _NP, _CW), jnp.float32),
        mesh=mesh,
        scratch_types=[
            [pltpu.VMEM((_K,), jnp.int32)] * NB,
            [pltpu.VMEM((_K,), jnp.int32)] * NB,
            [pltpu.VMEM((_K,), jnp.int32)] * NB,
            [pltpu.VMEM((_K * 16,), jnp.float32)] * NB,
            [pltpu.VMEM((_K, _CW), jnp.float32)] * NB,
            pltpu.VMEM_SHARED((_NP, _CW), jnp.float32),
            [pltpu.SemaphoreType.DMA] * NB,
            [pltpu.SemaphoreType.DMA] * NB,
            [pltpu.SemaphoreType.DMA] * NB,
            [pltpu.SemaphoreType.DMA] * NB,
            [pltpu.SemaphoreType.DMA] * NB,
        ],
    )
    def k(xwb_h, rr_h, cc_h, vn_h, z_h, out_h,
          ccb, idxb, rrb, vnb, rowsb, acc_sh,
          csem, gsem, vsem, rsem, ssem):
        c = lax.axis_index("c")
        s = lax.axis_index("s")
        ebase0 = s * epw

        for blk_i in range(bpc):
            b = c * bpc + blk_i
            roff = b * _NP

            pltpu.sync_copy(z_h, acc_sh.at[pl.ds(s * _SLICE, _SLICE)])
            plsc.subcore_barrier()

            def fire_meta(k_, j):
                eb = ebase0 + k_ * _K
                pltpu.async_copy(cc_h.at[pl.ds(eb, _K)], ccb[j], csem[j])
                pltpu.async_copy(vn_h.at[pl.ds(eb * 16, _K * 16)],
                                 vnb[j], vsem[j])
                pltpu.async_copy(rr_h.at[pl.ds(eb, _K)], rrb[j], rsem[j])

            def fire_gather(k_, j):
                pltpu.make_async_copy(cc_h.at[pl.ds(0, _K)], ccb[j],
                                      csem[j]).wait()
                for g in range(_K // 16):
                    idxb[j][pl.ds(g * 16, 16)] = (
                        ccb[j][pl.ds(g * 16, 16)] + roff)
                pltpu.async_copy(xwb_h.at[idxb[j]], rowsb[j], gsem[j])

            def wait_sc(j):
                pltpu.make_async_copy(rowsb[j], acc_sh.at[rrb[j]],
                                      ssem[j]).wait()

            def process(k_, j, do_meta, do_gather, do_wait_sc):
                pltpu.make_async_copy(xwb_h.at[idxb[j]], rowsb[j],
                                      gsem[j]).wait()
                if do_gather:
                    fire_gather(k_ + NB - 2, (j + NB - 2) % NB)
                pltpu.make_async_copy(vn_h.at[pl.ds(0, _K * 16)], vnb[j],
                                      vsem[j]).wait()

                def sb(e, cr):
                    bc = vnb[j][pl.ds(e * 16, 16)]
                    for q in range(_CW // 16):
                        rowsb[j][e, pl.ds(q * 16, 16)] = (
                            rowsb[j][e, pl.ds(q * 16, 16)] * bc)
                    return cr

                lax.fori_loop(0, _K, sb, 0, unroll=4)
                if do_meta:
                    if do_wait_sc:
                        wait_sc((j + NB - 1) % NB)
                    fire_meta(k_ + NB - 1, (j + NB - 1) % NB)
                pltpu.make_async_copy(rr_h.at[pl.ds(0, _K)], rrb[j],
                                      rsem[j]).wait()
                pltpu.async_copy(rowsb[j], acc_sh.at[rrb[j]], ssem[j],
                                 add=True)

            for k_ in range(NB - 1):
                fire_meta(k_, k_)
            for k_ in range(NB - 2):
                fire_gather(k_, k_)

            process(0, 0, True, True, False)
            for k_ in range(1, NB):
                process(k_, k_, True, True, True)

            def outer(m, cr):
                for i in range(NB):
                    process(m * NB + i, i, True, True, True)
                return cr

            lax.fori_loop(1, M - 1, outer, 0)

            base = (M - 1) * NB
            process(base + 0, 0, True, True, True)
            process(base + 1, 1, False, True, True)
            process(base + 2, 2, False, False, True)
            process(base + 3, 3, False, False, True)
            for j in range(NB):
                wait_sc(j)

            plsc.subcore_barrier()
            pltpu.sync_copy(acc_sh.at[pl.ds(s * _SLICE, _SLICE)],
                            out_h.at[pl.ds(roff + s * _SLICE, _SLICE)])

    return k(xwb, rr, cc, vn, zrows)


def kernel(x, W1, b1, W2, b2, W3, b3, Wp1, bp1, Wp2, bp2, edge_index,
           edge_values, training):
    n = x.shape[0]
    rr = edge_index[1]
    cc = edge_index[0]

    E0 = rr.shape[0]
    EP = -(-E0 // (_NS * _K * 4)) * (_NS * _K * 4)
    pad = EP - E0
    rrp = jnp.concatenate([rr, jnp.zeros((pad,), rr.dtype)])
    ccp = jnp.concatenate([cc, jnp.zeros((pad,), cc.dtype)])
    vvp = jnp.concatenate([edge_values, jnp.zeros((pad,), edge_values.dtype)])
    val16 = jnp.broadcast_to(vvp[:, None], (EP, 16)).reshape(EP * 16)

    deg = jax.ops.segment_sum(edge_values, rr, num_segments=n) + 1.0
    dn = jax.lax.rsqrt(deg)[:, None]

    zrows = jnp.zeros((_SLICE, _CW), jnp.float32)

    def agg(u):
        flat = _spmm_sc(u.reshape(_NBLK * _NP, _CW), rrp, ccp, val16, zrows)
        return flat.reshape(_NBLK, _NP, _CW)

    u1 = _first_matmul(x, W1, dn)
    a1 = agg(u1)
    u2 = _mid_matmul(a1, u1, dn, b1, W2)
    a2 = agg(u2)
    u3 = _mid_matmul(a2, u2, dn, b2, W3)
    a3 = agg(u3)
    emb, z = _final_stage(a3, u3, dn, b3, Wp1, bp1, Wp2, bp2)
    return (emb, z)


# speedup vs baseline: 1.0664x; 1.0664x over previous
import functools
import jax
import jax.numpy as jnp
from jax import lax
from jax.experimental import pallas as pl
from jax.experimental.pallas import tpu as pltpu
from jax.experimental.pallas import tpu_sc as plsc

_N = 10000
_NP = 10240
_D = 512
_CW = 128
_NBLK = _D // _CW
_NC = 2
_NS = 16
_SLICE = _NP // _NS
_K = 80

_BM = 1000


def _first_body(x_ref, w_ref, dn_ref, o_ref):
    xw = jnp.dot(x_ref[...], w_ref[...], preferred_element_type=jnp.float32)
    dn = dn_ref[...]
    for b in range(_NBLK):
        o_ref[b] = dn * xw[:, b * _CW:(b + 1) * _CW]


def _first_matmul(x, W, dn):
    M, K = x.shape
    return pl.pallas_call(
        _first_body,
        grid=(M // _BM,),
        in_specs=[
            pl.BlockSpec((_BM, K), lambda i: (i, 0)),
            pl.BlockSpec((K, _D), lambda i: (0, 0)),
            pl.BlockSpec((_BM, 1), lambda i: (i, 0)),
        ],
        out_specs=pl.BlockSpec((_NBLK, _BM, _CW), lambda i: (0, i, 0)),
        out_shape=jax.ShapeDtypeStruct((_NBLK, _NP, _CW), jnp.float32),
    )(x, W, dn)


def _mid_body(agg_ref, u_ref, dn_ref, b_ref, w_ref, o_ref):
    acc = jnp.zeros((_BM, _D), dtype=jnp.float32)
    dn = dn_ref[...]
    for b in range(_NBLK):
        sl = slice(b * _CW, (b + 1) * _CW)
        h = jnp.maximum(dn * (agg_ref[b] + u_ref[b]) + b_ref[:, sl], 0.0)
        acc = acc + jnp.dot(h, w_ref[sl, :], preferred_element_type=jnp.float32)
    for b in range(_NBLK):
        o_ref[b] = dn * acc[:, b * _CW:(b + 1) * _CW]


def _mid_matmul(agg, u, dn, bias, W):
    M = dn.shape[0]
    return pl.pallas_call(
        _mid_body,
        grid=(M // _BM,),
        in_specs=[
            pl.BlockSpec((_NBLK, _BM, _CW), lambda i: (0, i, 0)),
            pl.BlockSpec((_NBLK, _BM, _CW), lambda i: (0, i, 0)),
            pl.BlockSpec((_BM, 1), lambda i: (i, 0)),
            pl.BlockSpec((1, _D), lambda i: (0, 0)),
            pl.BlockSpec((_D, _D), lambda i: (0, 0)),
        ],
        out_specs=pl.BlockSpec((_NBLK, _BM, _CW), lambda i: (0, i, 0)),
        out_shape=jax.ShapeDtypeStruct((_NBLK, _NP, _CW), jnp.float32),
    )(agg, u, dn, bias.reshape(1, _D), W)


def _fin_body(agg_ref, u_ref, dn_ref, b3_ref, wp1_ref, bp1_ref, wp2_ref,
              bp2_ref, emb_ref, z_ref):
    dn = dn_ref[...]
    cols = []
    for b in range(_NBLK):
        sl = slice(b * _CW, (b + 1) * _CW)
        cols.append(dn * (agg_ref[b] + u_ref[b]) + b3_ref[:, sl])
    emb = jnp.concatenate(cols, axis=1)
    emb_ref[...] = emb
    t = jnp.maximum(
        jnp.dot(emb, wp1_ref[...], preferred_element_type=jnp.float32)
        + bp1_ref[...], 0.0)
    z_ref[...] = jnp.dot(t, wp2_ref[...],
                         preferred_element_type=jnp.float32) + bp2_ref[...]


def _final_stage(agg, u, dn, b3, Wp1, bp1, Wp2, bp2):
    M = dn.shape[0]
    P = Wp1.shape[1]
    return pl.pallas_call(
        _fin_body,
        grid=(M // _BM,),
        in_specs=[
            pl.BlockSpec((_NBLK, _BM, _CW), lambda i: (0, i, 0)),
            pl.BlockSpec((_NBLK, _BM, _CW), lambda i: (0, i, 0)),
            pl.BlockSpec((_BM, 1), lambda i: (i, 0)),
            pl.BlockSpec((1, _D), lambda i: (0, 0)),
            pl.BlockSpec((_D, P), lambda i: (0, 0)),
            pl.BlockSpec((1, P), lambda i: (0, 0)),
            pl.BlockSpec((P, P), lambda i: (0, 0)),
            pl.BlockSpec((1, P), lambda i: (0, 0)),
        ],
        out_specs=[
            pl.BlockSpec((_BM, _D), lambda i: (i, 0)),
            pl.BlockSpec((_BM, P), lambda i: (i, 0)),
        ],
        out_shape=[
            jax.ShapeDtypeStruct((M, _D), jnp.float32),
            jax.ShapeDtypeStruct((M, P), jnp.float32),
        ],
    )(agg, u, dn, b3.reshape(1, _D), Wp1, bp1.reshape(1, P), Wp2,
      bp2.reshape(1, P))


def _spmm_sc(xwb, meta, vn, zrows):
    MW = 2 * _K
    epw = meta.shape[0] // (_NS * 2)
    chunks = epw // _K
    assert chunks * _K == epw
    NB = 4
    assert chunks % NB == 0
    M = chunks // NB
    mesh = plsc.VectorSubcoreMesh(core_axis_name="c", subcore_axis_name="s",
                                  num_cores=_NC, num_subcores=_NS)
    bpc = _NBLK // _NC

    @functools.partial(
        pl.kernel,
        out_type=jax.ShapeDtypeStruct((_NBLK * _NP, _CW), jnp.float32),
        mesh=mesh,
        scratch_types=[
            [pltpu.VMEM((MW,), jnp.int32)] * NB,
            [pltpu.VMEM((_K,), jnp.int32)] * NB,
            [pltpu.VMEM((_K,), jnp.int32)] * NB,
            [pltpu.VMEM((_K * 16,), jnp.float32)] * NB,
            [pltpu.VMEM((_K, _CW), jnp.float32)] * NB,
            pltpu.VMEM_SHARED((_NP, _CW), jnp.float32),
            [pltpu.SemaphoreType.DMA] * NB,
            [pltpu.SemaphoreType.DMA] * NB,
            [pltpu.SemaphoreType.DMA] * NB,
            [pltpu.SemaphoreType.DMA] * NB,
        ],
    )
    def k(xwb_h, meta_h, vn_h, z_h, out_h,
          mb, idxb, rrb, vnb, rowsb, acc_sh, msem, vsem, gsem, ssem):
        c = lax.axis_index("c")
        s = lax.axis_index("s")
        mbase0 = s * chunks * MW
        ebase0 = s * epw

        for blk_i in range(bpc):
            b = c * bpc + blk_i
            roff = b * _NP

            pltpu.sync_copy(z_h, acc_sh.at[pl.ds(s * _SLICE, _SLICE)])
            plsc.subcore_barrier()

            def fire_meta(k_, j):
                pltpu.async_copy(meta_h.at[pl.ds(mbase0 + k_ * MW, MW)],
                                 mb[j], msem[j])
                pltpu.async_copy(vn_h.at[pl.ds((ebase0 + k_ * _K) * 16,
                                               _K * 16)], vnb[j], vsem[j])

            def fire_gather(j):
                pltpu.make_async_copy(meta_h.at[pl.ds(0, MW)], mb[j],
                                      msem[j]).wait()
                for g in range(_K // 16):
                    idxb[j][pl.ds(g * 16, 16)] = (
                        mb[j][pl.ds(g * 16, 16)] + roff)
                pltpu.async_copy(xwb_h.at[idxb[j]], rowsb[j], gsem[j])

            def wait_sc(j):
                pltpu.make_async_copy(rowsb[j], acc_sh.at[rrb[j]],
                                      ssem[j]).wait()

            def process(k_, j, do_meta, do_gather, do_wait_sc):
                pltpu.make_async_copy(xwb_h.at[idxb[j]], rowsb[j],
                                      gsem[j]).wait()
                for g in range(_K // 16):
                    rrb[j][pl.ds(g * 16, 16)] = mb[j][pl.ds(_K + g * 16, 16)]
                pltpu.make_async_copy(vn_h.at[pl.ds(0, _K * 16)], vnb[j],
                                      vsem[j]).wait()

                def sb(e, cr):
                    bc = vnb[j][pl.ds(e * 16, 16)]
                    for q in range(_CW // 16):
                        rowsb[j][e, pl.ds(q * 16, 16)] = (
                            rowsb[j][e, pl.ds(q * 16, 16)] * bc)
                    return cr

                lax.fori_loop(0, _K, sb, 0, unroll=4)
                pltpu.async_copy(rowsb[j], acc_sh.at[rrb[j]], ssem[j],
                                 add=True)
                if do_meta:
                    if do_wait_sc:
                        wait_sc((j + NB - 1) % NB)
                    fire_meta(k_ + NB - 1, (j + NB - 1) % NB)
                if do_gather:
                    fire_gather((j + NB - 2) % NB)

            for k_ in range(NB - 1):
                fire_meta(k_, k_)
            for k_ in range(NB - 2):
                fire_gather(k_)

            process(0, 0, True, True, False)
            for k_ in range(1, NB):
                process(k_, k_, True, True, True)

            def outer(m, cr):
                for i in range(NB):
                    process(m * NB + i, i, True, True, True)
                return cr

            lax.fori_loop(1, M - 1, outer, 0)

            base = (M - 1) * NB
            process(base + 0, 0, True, True, True)
            process(base + 1, 1, False, True, True)
            process(base + 2, 2, False, False, True)
            process(base + 3, 3, False, False, True)
            for j in range(NB):
                wait_sc(j)

            plsc.subcore_barrier()
            pltpu.sync_copy(acc_sh.at[pl.ds(s * _SLICE, _SLICE)],
                            out_h.at[pl.ds(roff + s * _SLICE, _SLICE)])

    return k(xwb, meta, vn, zrows)


def kernel(x, W1, b1, W2, b2, W3, b3, Wp1, bp1, Wp2, bp2, edge_index,
           edge_values, training):
    n = x.shape[0]
    rr = edge_index[1]
    cc = edge_index[0]

    E0 = rr.shape[0]
    EP = -(-E0 // (_NS * _K * 4)) * (_NS * _K * 4)
    pad = EP - E0
    rrp = jnp.concatenate([rr, jnp.zeros((pad,), rr.dtype)])
    ccp = jnp.concatenate([cc, jnp.zeros((pad,), cc.dtype)])
    vvp = jnp.concatenate([edge_values, jnp.zeros((pad,), edge_values.dtype)])
    val16 = jnp.broadcast_to(vvp[:, None], (EP, 16)).reshape(EP * 16)

    chunks = EP // _NS // _K
    meta = jnp.concatenate(
        [ccp.reshape(_NS, chunks, _K),
         rrp.reshape(_NS, chunks, _K)], axis=2).reshape(-1)

    deg = jax.ops.segment_sum(edge_values, rr, num_segments=n) + 1.0
    dn = jax.lax.rsqrt(deg)[:, None]

    zrows = jnp.zeros((_SLICE, _CW), jnp.float32)

    def agg(u):
        flat = _spmm_sc(u.reshape(_NBLK * _NP, _CW), meta, val16, zrows)
        return flat.reshape(_NBLK, _NP, _CW)

    u1 = _first_matmul(x, W1, dn)
    a1 = agg(u1)
    u2 = _mid_matmul(a1, u1, dn, b1, W2)
    a2 = agg(u2)
    u3 = _mid_matmul(a2, u2, dn, b2, W3)
    a3 = agg(u3)
    emb, z = _final_stage(a3, u3, dn, b3, Wp1, bp1, Wp2, bp2)
    return (emb, z)
